# baseline (speedup 1.0000x reference)
import jax
import jax.numpy as jnp
from jax import lax
from jax.experimental import pallas as pl
from jax.experimental.pallas import tpu as pltpu
from jax.experimental.pallas import tpu_sc as plsc

_B, _P, _C, _H, _W = 4, 14, 192, 16, 16
_NPATCH = _B * _P * _P
_CK = 16
_NCHUNK = _C // _CK
_JOBS = _NPATCH * _NCHUNK
_NC, _NS = 2, 16
_NW = _NC * _NS
_JPW = _JOBS // _NW
_XT = _H * _W
_OT = (_H + 2) * (_W + 2)


def _body(xf, tw, bw, lw, rw, tlw, trw, blw, brw, of, wts, xbuf, obuf):
    wid = lax.axis_index("s") * _NC + lax.axis_index("c")
    pltpu.sync_copy(tw, wts.at[0])
    pltpu.sync_copy(bw, wts.at[1])
    pltpu.sync_copy(lw, wts.at[2])
    pltpu.sync_copy(rw, wts.at[3])
    pltpu.sync_copy(tlw, wts.at[4])
    pltpu.sync_copy(trw, wts.at[5])
    pltpu.sync_copy(blw, wts.at[6])
    pltpu.sync_copy(brw, wts.at[7])

    lanes = lax.iota(jnp.int32, 16)
    i16 = lanes * 16
    i18 = lanes * 18
    i256 = lanes * _XT
    i324 = lanes * _OT

    def job(j, carry):
        jg = wid * _JPW + j
        b = jg // _NCHUNK
        c0 = (jg % _NCHUNK) * _CK
        pr = (b % (_P * _P)) // _P
        pc = b % _P
        one = jnp.float32(1.0)
        zero = jnp.float32(0.0)
        mt = jnp.where(pr == 0, zero, one)
        mb = jnp.where(pr == _P - 1, zero, one)
        ml = jnp.where(pc == 0, zero, one)
        mr = jnp.where(pc == _P - 1, zero, one)

        pltpu.sync_copy(xf.at[pl.ds(b * (_C * _XT) + c0 * _XT, _CK * _XT)],
                        xbuf)

        twv = wts[0, pl.ds(c0, _CK)] * mt
        bwv = wts[1, pl.ds(c0, _CK)] * mb
        lwv = wts[2, pl.ds(c0, _CK)] * ml
        rwv = wts[3, pl.ds(c0, _CK)] * mr

        for k in range(_CK):
            xk = _XT * k
            ok = _OT * k
            rsave = {}
            for h in range(_H):
                r = xbuf[pl.ds(xk + 16 * h, 16)]
                plsc.store_scatter(obuf, [lanes + (ok + 18 * (h + 1) + 1)], r)
                if h in (0, 1, _H - 2, _H - 1):
                    rsave[h] = r
            plsc.store_scatter(obuf, [lanes + (ok + 1)],
                               twv[k] * (rsave[0] + rsave[1]))
            plsc.store_scatter(obuf, [lanes + (ok + 18 * (_H + 1) + 1)],
                               bwv[k] * (rsave[_H - 2] + rsave[_H - 1]))
            wlk = lwv[k]
            wrk = rwv[k]
            g0 = plsc.load_gather(xbuf, [i16 + xk])
            g1 = plsc.load_gather(xbuf, [i16 + (xk + 1)])
            plsc.store_scatter(obuf, [i18 + (ok + 18)], wlk * (g0 + g1))
            g0 = plsc.load_gather(xbuf, [i16 + (xk + _W - 2)])
            g1 = plsc.load_gather(xbuf, [i16 + (xk + _W - 1)])
            plsc.store_scatter(obuf, [i18 + (ok + 35)], wrk * (g0 + g1))

        tlv = wts[4, pl.ds(c0, _CK)] * (mt * ml)
        trv = wts[5, pl.ds(c0, _CK)] * (mt * mr)
        blv = wts[6, pl.ds(c0, _CK)] * (mb * ml)
        brv = wts[7, pl.ds(c0, _CK)] * (mb * mr)
        g = plsc.load_gather(xbuf, [i256])
        plsc.store_scatter(obuf, [i324], tlv * g)
        g = plsc.load_gather(xbuf, [i256 + (_W - 1)])
        plsc.store_scatter(obuf, [i324 + (_W + 1)], trv * g)
        g = plsc.load_gather(xbuf, [i256 + (_XT - _W)])
        plsc.store_scatter(obuf, [i324 + 18 * (_H + 1)], blv * g)
        g = plsc.load_gather(xbuf, [i256 + (_XT - 1)])
        plsc.store_scatter(obuf, [i324 + (_OT - 1)], brv * g)

        pltpu.sync_copy(obuf,
                        of.at[pl.ds(b * (_C * _OT) + c0 * _OT, _CK * _OT)])
        return carry

    lax.fori_loop(0, _JPW, job, 0)


def kernel(x, topW, botW, leftW, rightW, topleftW, toprightW, botleftW,
           botrightW):
    mesh = plsc.VectorSubcoreMesh(core_axis_name="c", subcore_axis_name="s",
                                  num_cores=_NC, num_subcores=_NS)
    f = pl.kernel(
        _body,
        out_type=jax.ShapeDtypeStruct((_NPATCH * _C * _OT,), jnp.float32),
        mesh=mesh,
        compiler_params=pltpu.CompilerParams(needs_layout_passes=False),
        scratch_types=[
            pltpu.VMEM((8, _C), jnp.float32),
            pltpu.VMEM((_CK * _XT,), jnp.float32),
            pltpu.VMEM((_CK * _OT,), jnp.float32),
        ],
    )
    xflat = x.reshape(_NPATCH * _C * _XT)
    out = f(xflat, topW, botW, leftW, rightW, topleftW, toprightW, botleftW,
            botrightW)
    return out.reshape(_NPATCH, _C, _H + 2, _W + 2)
